# baseline (speedup 1.0000x reference)
import functools
import math

import jax
import jax.numpy as jnp
from jax import lax
from jax.experimental import pallas as pl
from jax.experimental.pallas import tpu as pltpu
from jax.experimental.pallas import tpu_sc as plsc

TOPK = 13
NC = 80
EPS_IN = 1e-9
EPS_IOU = 1e-7

_ATAN_COEFFS = (
    9.999999999776e-01, -3.333333207911e-01, 1.999992819090e-01,
    -1.428419858875e-01, 1.109481209099e-01, -8.987009293820e-02,
    7.264201452628e-02, -5.460683193113e-02, 3.458600582713e-02,
    -1.638182583930e-02, 4.961051519167e-03, -7.042539361997e-04,
)


def _atan_pos(x):
    inv = x > 1.0
    z = jnp.where(inv, 1.0 / x, x)
    u = z * z
    p = jnp.full_like(u, _ATAN_COEFFS[-1])
    for c in _ATAN_COEFFS[-2::-1]:
        p = p * u + c
    at = z * p
    return jnp.where(inv, (math.pi / 2) - at, at)


def _align_body(ps_ref, pb_ref, anc_ref, gl_ref, gtb_ref, gm_ref,
                align_ref, ncm_ref):
    ngt = gtb_ref.shape[1]
    na = pb_ref.shape[2]

    pb = pb_ref[0]
    px1 = pb[0:1, :]
    px2 = pb[1:2, :]
    py1 = pb[2:3, :]
    py2 = pb[3:4, :]
    anc = anc_ref[0]
    ax = anc[0:1, :]
    ay = anc[1:2, :]
    gtb = gtb_ref[0]
    gx1 = gtb[:, 0:1]
    gx2 = gtb[:, 1:2]
    gy1 = gtb[:, 2:3]
    gy2 = gtb[:, 3:4]

    d1 = ax - gx1
    d2 = ay - gx2
    d3 = gy1 - ax
    d4 = gy2 - ay
    dmin = jnp.minimum(jnp.minimum(d1, d2), jnp.minimum(d3, d4))
    valid = gm_ref[0] > 0.0
    mask = jnp.logical_and(dmin > EPS_IN, valid)

    inter = (jnp.clip(jnp.minimum(px2, gx2) - jnp.maximum(px1, gx1), 0.0, None)
             * jnp.clip(jnp.minimum(py2, gy2) - jnp.maximum(py1, gy1), 0.0, None))
    w1 = px2 - px1
    h1 = py2 - py1 + EPS_IOU
    w2 = gx2 - gx1
    h2 = gy2 - gy1 + EPS_IOU
    union = w1 * h1 + w2 * h2 - inter + EPS_IOU
    iou = inter / union
    cw = jnp.maximum(px2, gx2) - jnp.minimum(px1, gx1)
    ch = jnp.maximum(py2, gy2) - jnp.minimum(py1, gy1)
    c2 = cw * cw + ch * ch + EPS_IOU
    rho2 = ((gx1 + gx2 - px1 - px2) ** 2 + (gy1 + gy2 - py1 - py2) ** 2) / 4.0
    at1 = _atan_pos(w1 / h1)
    at2 = _atan_pos(w2 / h2)
    dat = at2 - at1
    v = (4.0 / math.pi ** 2) * dat * dat
    alpha = v / (v - iou + (1.0 + EPS_IOU))
    ciou = iou - (rho2 / c2 + v * alpha)
    iou_c = jnp.maximum(ciou, 0.0)

    ps = ps_ref[0]
    gl = gl_ref[0]
    cls_iota = jax.lax.broadcasted_iota(jnp.int32, ps.shape, 1)
    oh = (cls_iota == gl).astype(ps.dtype)
    s = jnp.sum(ps * oh, axis=1, keepdims=True)

    i2 = iou_c * iou_c
    i6 = i2 * i2 * i2
    align = s * i6 * mask.astype(ps.dtype)
    align_ref[0] = align

    i_iota = jax.lax.broadcasted_iota(jnp.int32, (ngt, na), 1)
    j_iota = jax.lax.broadcasted_iota(jnp.int32, (ngt, na), 0)
    amax = jnp.max(align, axis=0, keepdims=True)
    cg = jnp.min(jnp.where(align == amax, j_iota, ngt), axis=0,
                 keepdims=True)
    hit = jnp.max((cg == j_iota).astype(jnp.int32), axis=1,
                  keepdims=True)
    ncm_row = jnp.max(jnp.where(i_iota == j_iota, hit, 0), axis=0,
                      keepdims=True)
    ncm_ref[0] = jnp.broadcast_to(ncm_row, (8, na))


def _sc_topk_body(align_hbm, bmp_hbm, buf0, buf1, bmp0, bmp1, cvals, cidx,
                  isem0, isem1, osem0, osem1):
    nrows, na = align_hbm.shape
    nchunk = na // 16
    ngrp = nchunk // 16
    rpw = nrows // 32
    wid = lax.axis_index("s") * 2 + lax.axis_index("c")
    lane = lax.iota(jnp.int32, 16)
    bufs = (buf0, buf1)
    bmps = (bmp0, bmp1)
    isems = (isem0, isem1)
    osems = (osem0, osem1)

    def zloop(k, c):
        bmp0[pl.ds(k * 16, 16)] = jnp.zeros((16,), jnp.int32)
        bmp1[pl.ds(k * 16, 16)] = jnp.zeros((16,), jnp.int32)
        return c

    lax.fori_loop(0, nchunk, zloop, 0)

    def process_row(buf):
        def grp_body(g, off):
            vs, ms, pcs = [], [], []
            for j in range(16):
                v = buf[pl.ds((g * 16 + j) * 16, 16)]
                m = v > 1e-9
                vs.append(v)
                ms.append(m)
                pcs.append(plsc.all_reduce_population_count(m))
            o = off
            for j in range(16):
                plsc.store_compressed(cvals.at[pl.ds(o, 16)], vs[j],
                                      mask=ms[j])
                plsc.store_compressed(cidx.at[pl.ds(o, 16)],
                                      lane + (g * 16 + j) * 16, mask=ms[j])
                o = o + pcs[j][0]
            return o

        cnt = lax.fori_loop(0, ngrp, grp_body, jnp.int32(0))

        def tail_body(k, off):
            v = buf[pl.ds(k * 16, 16)]
            m = v > 1e-9
            pc = plsc.all_reduce_population_count(m)
            plsc.store_compressed(cvals.at[pl.ds(off, 16)], v, mask=m)
            plsc.store_compressed(cidx.at[pl.ds(off, 16)], lane + k * 16,
                                  mask=m)
            return off + pc[0]

        cnt = lax.fori_loop(ngrp * 16, nchunk, tail_body, cnt)

        def cand_body(g, carry):
            cv, ci = carry
            base = g * 16
            inb = (base + lane) < cnt
            v = jnp.where(inb, cvals[pl.ds(base, 16)], -jnp.inf)
            iv = cidx[pl.ds(base, 16)]
            sv, si = plsc.sort_key_val(v, iv, descending=True)
            svr = lax.rev(sv, (0,))
            sir = lax.rev(si, (0,))
            pick = cv >= svr
            nv = jnp.where(pick, cv, svr)
            ni = jnp.where(pick, ci, sir)
            nv2, ni2 = plsc.sort_key_val(nv, ni, descending=True)
            return nv2, ni2

        cv0 = jnp.full((16,), -jnp.inf, jnp.float32)
        ci0 = jnp.zeros((16,), jnp.int32)
        cv, ci = lax.fori_loop(0, (cnt + 15) // 16, cand_body, (cv0, ci0))
        return cv, ci

    row0 = wid * rpw
    pending_out = [None, None]
    in_copies = [None, None]
    in_copies[0] = pltpu.make_async_copy(align_hbm.at[row0], bufs[0],
                                         isems[0])
    in_copies[0].start()
    for t in range(rpw):
        cur = t % 2
        nxt = (t + 1) % 2
        if t + 1 < rpw:
            in_copies[nxt] = pltpu.make_async_copy(
                align_hbm.at[row0 + t + 1], bufs[nxt], isems[nxt])
            in_copies[nxt].start()
        in_copies[cur].wait()
        cv, ci = process_row(bufs[cur])
        msk = jnp.logical_and(cv > 1e-9, lane < TOPK)
        if pending_out[cur] is not None:
            oc, oci, omsk = pending_out[cur]
            oc.wait()
            plsc.store_scatter(bmps[cur], [oci],
                               jnp.zeros((16,), jnp.int32), mask=omsk)
        plsc.store_scatter(bmps[cur], [ci], jnp.ones((16,), jnp.int32),
                           mask=msk)
        oc = pltpu.make_async_copy(bmps[cur], bmp_hbm.at[row0 + t],
                                   osems[cur])
        oc.start()
        pending_out[cur] = (oc, ci, msk)
    for cur in range(2):
        if pending_out[cur] is not None:
            pending_out[cur][0].wait()


def _out_body(bmpa_ref, bmpb_ref, ncma_ref, ncmb_ref, gl_ref, gtb_ref,
              cls_ref, tb_ref, ts_ref, fm_ref, tg_ref, cfl_ref):
    b = pl.program_id(0)
    bs = pl.num_programs(0)
    hb = bs // 2
    ngt = gtb_ref.shape[1]
    na = bmpa_ref.shape[1]

    @pl.when(b == 0)
    def _flags():
        acf = jnp.int32(0)
        for bb in range(bs):
            ref = bmpa_ref if bb < hb else bmpb_ref
            isb = ref[pl.ds((bb % hb) * ngt, ngt), :]
            cntb = jnp.sum(isb, axis=0, keepdims=True)
            acf = acf + jnp.max((cntb > 1).astype(jnp.int32))
        cfl_ref[0] = acf

    in_a = b < hb
    idx = jnp.where(in_a, b, b - hb)
    a_sl = bmpa_ref[pl.ds(idx * ngt, ngt), :]
    b_sl = bmpb_ref[pl.ds(idx * ngt, ngt), :]
    is_in = jnp.where(in_a, a_sl, b_sl)
    cnt = jnp.sum(is_in, axis=0, keepdims=True)
    conflict = cnt > 1

    gl = gl_ref[0]
    gtb = gtb_ref[0]
    j_iota = jax.lax.broadcasted_iota(jnp.int32, (ngt, na), 0)
    ncm_row = jnp.where(in_a, ncma_ref[0][0:1, :],
                        ncmb_ref[0][0:1, :])
    nm = jnp.logical_not(conflict).astype(jnp.int32)
    ncm = jnp.where(j_iota == 0, ncm_row, 0)
    resolved = (is_in + ncm) * nm
    anycf = cfl_ref[0]
    is_f = jnp.where(anycf > 0, resolved, is_in)

    fmask = jnp.sum(is_f, axis=0, keepdims=True) > 0
    mxv = jnp.max(is_f, axis=0, keepdims=True)
    tg = jnp.min(jnp.where(is_f == mxv, j_iota, ngt), axis=0,
                 keepdims=True)

    sel_t = j_iota == tg
    cmp_t = sel_t.astype(jnp.float32)
    glf = gl.astype(jnp.float32)
    tcls = jnp.sum(jnp.where(sel_t, glf, 0.0), axis=0, keepdims=True)

    cls_ref[0] = tcls.astype(jnp.int32)
    fm_ref[0] = fmask.astype(jnp.int32)
    tg_ref[0] = tg

    dn = (((0,), (0,)), ((), ()))
    tb_ref[0] = jax.lax.dot_general(cmp_t, gtb, dn,
                                    preferred_element_type=jnp.float32)
    oh_cls = (jax.lax.broadcasted_iota(jnp.int32, (ngt, NC), 1)
              == gl).astype(jnp.float32)
    ts_ref[0] = jax.lax.dot_general(cmp_t, oh_cls, dn,
                                    preferred_element_type=jnp.float32)


@jax.jit
def kernel(pd_scores, pd_bboxes, anc_points, gt_labels, gt_bboxes, gt_mask):
    bs, na, nc = pd_scores.shape
    ngt = gt_bboxes.shape[1]

    ps_s = pd_scores[:, :ngt, :]
    pb_t = jnp.transpose(pd_bboxes, (0, 2, 1))
    anc_t = jnp.transpose(anc_points).reshape(1, 2, na)
    gl = gt_labels.astype(jnp.int32)

    def tc_align(ps_h, pb_h, gl_h, gtb_h, gm_h):
        bh = ps_h.shape[0]
        return pl.pallas_call(
            _align_body,
            grid=(bh,),
            in_specs=[
                pl.BlockSpec((1, ngt, nc), lambda b: (b, 0, 0)),
                pl.BlockSpec((1, 4, na), lambda b: (b, 0, 0)),
                pl.BlockSpec((1, 2, na), lambda b: (0, 0, 0)),
                pl.BlockSpec((1, ngt, 1), lambda b: (b, 0, 0)),
                pl.BlockSpec((1, ngt, 4), lambda b: (b, 0, 0)),
                pl.BlockSpec((1, ngt, 1), lambda b: (b, 0, 0)),
            ],
            out_specs=(
                pl.BlockSpec((1, ngt, na), lambda b: (b, 0, 0)),
                pl.BlockSpec((1, 8, na), lambda b: (b, 0, 0)),
            ),
            out_shape=(
                jax.ShapeDtypeStruct((bh, ngt, na), jnp.float32),
                jax.ShapeDtypeStruct((bh, 8, na), jnp.int32),
            ),
        )(ps_h, pb_h, anc_t, gl_h, gtb_h, gm_h)

    mesh = plsc.VectorSubcoreMesh(core_axis_name="c", subcore_axis_name="s",
                                  num_cores=2, num_subcores=16)

    def sc_topk(align_h):
        return pl.kernel(
            _sc_topk_body,
            out_type=jax.ShapeDtypeStruct(align_h.shape, jnp.int32),
            mesh=mesh,
            compiler_params=pltpu.CompilerParams(needs_layout_passes=False),
            scratch_types=[
                pltpu.VMEM((na,), jnp.float32),
                pltpu.VMEM((na,), jnp.float32),
                pltpu.VMEM((na,), jnp.int32),
                pltpu.VMEM((na,), jnp.int32),
                pltpu.VMEM((na + 16,), jnp.float32),
                pltpu.VMEM((na + 16,), jnp.int32),
                pltpu.SemaphoreType.DMA,
                pltpu.SemaphoreType.DMA,
                pltpu.SemaphoreType.DMA,
                pltpu.SemaphoreType.DMA,
            ],
        )(align_h)

    hb = bs // 2
    align_a, ncm_a = tc_align(ps_s[:hb], pb_t[:hb], gl[:hb],
                              gt_bboxes[:hb], gt_mask[:hb])
    bmp_a = sc_topk(align_a.reshape(hb * ngt, na))
    align_b, ncm_b = tc_align(ps_s[hb:], pb_t[hb:], gl[hb:],
                              gt_bboxes[hb:], gt_mask[hb:])
    bmp_b = sc_topk(align_b.reshape(hb * ngt, na))

    out_shapes = (
        jax.ShapeDtypeStruct((bs, 1, na), jnp.int32),
        jax.ShapeDtypeStruct((bs, na, 4), jnp.float32),
        jax.ShapeDtypeStruct((bs, na, NC), jnp.float32),
        jax.ShapeDtypeStruct((bs, 1, na), jnp.int32),
        jax.ShapeDtypeStruct((bs, 1, na), jnp.int32),
    )

    tcls, tb, ts, fm, tg = pl.pallas_call(
        _out_body,
        grid=(bs,),
        in_specs=[
            pl.BlockSpec((hb * ngt, na), lambda b: (0, 0)),
            pl.BlockSpec((hb * ngt, na), lambda b: (0, 0)),
            pl.BlockSpec((1, 8, na),
                         lambda b: (jnp.where(b < hb, b, b - hb), 0, 0)),
            pl.BlockSpec((1, 8, na),
                         lambda b: (jnp.where(b < hb, b, b - hb), 0, 0)),
            pl.BlockSpec((1, ngt, 1), lambda b: (b, 0, 0)),
            pl.BlockSpec((1, ngt, 4), lambda b: (b, 0, 0)),
        ],
        out_specs=(
            pl.BlockSpec((1, 1, na), lambda b: (b, 0, 0)),
            pl.BlockSpec((1, na, 4), lambda b: (b, 0, 0)),
            pl.BlockSpec((1, na, NC), lambda b: (b, 0, 0)),
            pl.BlockSpec((1, 1, na), lambda b: (b, 0, 0)),
            pl.BlockSpec((1, 1, na), lambda b: (b, 0, 0)),
        ),
        out_shape=out_shapes,
        scratch_shapes=[
            pltpu.SMEM((1,), jnp.int32),
        ],
    )(bmp_a, bmp_b, ncm_a, ncm_b, gl, gt_bboxes)

    return (tcls.reshape(bs, na), tb, ts,
            fm.reshape(bs, na) > 0, tg.reshape(bs, na))


# speedup vs baseline: 1.0018x; 1.0018x over previous
import math

import jax
import jax.numpy as jnp
from jax import lax
from jax.experimental import pallas as pl
from jax.experimental.pallas import tpu as pltpu
from jax.experimental.pallas import tpu_sc as plsc

TOPK = 13
NC = 80
EPS_IN = 1e-9
EPS_IOU = 1e-7

_ATAN_COEFFS = (
    9.999999999776e-01, -3.333333207911e-01, 1.999992819090e-01,
    -1.428419858875e-01, 1.109481209099e-01, -8.987009293820e-02,
    7.264201452628e-02, -5.460683193113e-02, 3.458600582713e-02,
    -1.638182583930e-02, 4.961051519167e-03, -7.042539361997e-04,
)


def _atan_pos(x):
    inv = x > 1.0
    z = jnp.where(inv, 1.0 / x, x)
    u = z * z
    p = jnp.full_like(u, _ATAN_COEFFS[-1])
    for c in _ATAN_COEFFS[-2::-1]:
        p = p * u + c
    at = z * p
    return jnp.where(inv, (math.pi / 2) - at, at)


def _align_body(ps_ref, pb_ref, anc_ref, gl_ref, gtb_ref, gm_ref,
                align_ref, ncm_ref):
    ngt = gtb_ref.shape[1]
    na = pb_ref.shape[2]

    pb = pb_ref[0]
    px1 = pb[0:1, :]
    px2 = pb[1:2, :]
    py1 = pb[2:3, :]
    py2 = pb[3:4, :]
    anc = anc_ref[0]
    ax = anc[0:1, :]
    ay = anc[1:2, :]
    gtb = gtb_ref[0]
    gx1 = gtb[:, 0:1]
    gx2 = gtb[:, 1:2]
    gy1 = gtb[:, 2:3]
    gy2 = gtb[:, 3:4]

    d1 = ax - gx1
    d2 = ay - gx2
    d3 = gy1 - ax
    d4 = gy2 - ay
    dmin = jnp.minimum(jnp.minimum(d1, d2), jnp.minimum(d3, d4))
    valid = gm_ref[0] > 0.0
    mask = jnp.logical_and(dmin > EPS_IN, valid)

    inter = (jnp.clip(jnp.minimum(px2, gx2) - jnp.maximum(px1, gx1), 0.0, None)
             * jnp.clip(jnp.minimum(py2, gy2) - jnp.maximum(py1, gy1), 0.0, None))
    w1 = px2 - px1
    h1 = py2 - py1 + EPS_IOU
    w2 = gx2 - gx1
    h2 = gy2 - gy1 + EPS_IOU
    union = w1 * h1 + w2 * h2 - inter + EPS_IOU
    iou = inter / union
    cw = jnp.maximum(px2, gx2) - jnp.minimum(px1, gx1)
    ch = jnp.maximum(py2, gy2) - jnp.minimum(py1, gy1)
    c2 = cw * cw + ch * ch + EPS_IOU
    rho2 = ((gx1 + gx2 - px1 - px2) ** 2 + (gy1 + gy2 - py1 - py2) ** 2) / 4.0
    at1 = _atan_pos(w1 / h1)
    at2 = _atan_pos(w2 / h2)
    dat = at2 - at1
    v = (4.0 / math.pi ** 2) * dat * dat
    alpha = v / (v - iou + (1.0 + EPS_IOU))
    ciou = iou - (rho2 / c2 + v * alpha)
    iou_c = jnp.maximum(ciou, 0.0)

    ps = ps_ref[0]
    gl = gl_ref[0]
    cls_iota = jax.lax.broadcasted_iota(jnp.int32, ps.shape, 1)
    oh = (cls_iota == gl).astype(ps.dtype)
    s = jnp.sum(ps * oh, axis=1, keepdims=True)

    i2 = iou_c * iou_c
    i6 = i2 * i2 * i2
    align = s * i6 * mask.astype(ps.dtype)
    align_ref[0] = align

    i_iota = jax.lax.broadcasted_iota(jnp.int32, (ngt, na), 1)
    j_iota = jax.lax.broadcasted_iota(jnp.int32, (ngt, na), 0)
    amax = jnp.max(align, axis=0, keepdims=True)
    cg = jnp.min(jnp.where(align == amax, j_iota, ngt), axis=0,
                 keepdims=True)
    hit = jnp.max((cg == j_iota).astype(jnp.int32), axis=1,
                  keepdims=True)
    ncm_row = jnp.max(jnp.where(i_iota == j_iota, hit, 0), axis=0,
                      keepdims=True)
    ncm_ref[0] = jnp.broadcast_to(ncm_row, (8, na))


def _sc_topk_body(align_hbm, bmp_hbm, buf0, buf1, bmp0, bmp1, cvals, cidx,
                  isem0, isem1, osem0, osem1):
    nrows, na = align_hbm.shape
    nchunk = na // 16
    ngrp = nchunk // 16
    rpw = nrows // 32
    wid = lax.axis_index("s") * 2 + lax.axis_index("c")
    lane = lax.iota(jnp.int32, 16)
    bufs = (buf0, buf1)
    bmps = (bmp0, bmp1)
    isems = (isem0, isem1)
    osems = (osem0, osem1)

    def zloop(k, c):
        bmp0[pl.ds(k * 16, 16)] = jnp.zeros((16,), jnp.int32)
        bmp1[pl.ds(k * 16, 16)] = jnp.zeros((16,), jnp.int32)
        return c

    lax.fori_loop(0, nchunk, zloop, 0)

    def process_row(buf):
        def grp_body(g, off):
            vs, ms, pcs = [], [], []
            for j in range(16):
                v = buf[pl.ds((g * 16 + j) * 16, 16)]
                m = v > 1e-9
                vs.append(v)
                ms.append(m)
                pcs.append(plsc.all_reduce_population_count(m))
            o = off
            for j in range(16):
                plsc.store_compressed(cvals.at[pl.ds(o, 16)], vs[j],
                                      mask=ms[j])
                plsc.store_compressed(cidx.at[pl.ds(o, 16)],
                                      lane + (g * 16 + j) * 16, mask=ms[j])
                o = o + pcs[j][0]
            return o

        cnt = lax.fori_loop(0, ngrp, grp_body, jnp.int32(0))

        def tail_body(k, off):
            v = buf[pl.ds(k * 16, 16)]
            m = v > 1e-9
            pc = plsc.all_reduce_population_count(m)
            plsc.store_compressed(cvals.at[pl.ds(off, 16)], v, mask=m)
            plsc.store_compressed(cidx.at[pl.ds(off, 16)], lane + k * 16,
                                  mask=m)
            return off + pc[0]

        cnt = lax.fori_loop(ngrp * 16, nchunk, tail_body, cnt)

        def cand_body(g, carry):
            cv, ci = carry
            base = g * 16
            inb = (base + lane) < cnt
            v = jnp.where(inb, cvals[pl.ds(base, 16)], -jnp.inf)
            iv = cidx[pl.ds(base, 16)]
            sv, si = plsc.sort_key_val(v, iv, descending=True)
            svr = lax.rev(sv, (0,))
            sir = lax.rev(si, (0,))
            pick = cv >= svr
            nv = jnp.where(pick, cv, svr)
            ni = jnp.where(pick, ci, sir)
            nv2, ni2 = plsc.sort_key_val(nv, ni, descending=True)
            return nv2, ni2

        cv0 = jnp.full((16,), -jnp.inf, jnp.float32)
        ci0 = jnp.zeros((16,), jnp.int32)
        cv, ci = lax.fori_loop(0, (cnt + 15) // 16, cand_body, (cv0, ci0))
        return cv, ci

    row0 = wid * rpw
    pending_out = [None, None]
    in_copies = [None, None]
    in_copies[0] = pltpu.make_async_copy(align_hbm.at[row0], bufs[0],
                                         isems[0])
    in_copies[0].start()
    for t in range(rpw):
        cur = t % 2
        nxt = (t + 1) % 2
        if t + 1 < rpw:
            in_copies[nxt] = pltpu.make_async_copy(
                align_hbm.at[row0 + t + 1], bufs[nxt], isems[nxt])
            in_copies[nxt].start()
        in_copies[cur].wait()
        cv, ci = process_row(bufs[cur])
        msk = jnp.logical_and(cv > 1e-9, lane < TOPK)
        if pending_out[cur] is not None:
            oc, oci, omsk = pending_out[cur]
            oc.wait()
            plsc.store_scatter(bmps[cur], [oci],
                               jnp.zeros((16,), jnp.int32), mask=omsk)
        plsc.store_scatter(bmps[cur], [ci], jnp.ones((16,), jnp.int32),
                           mask=msk)
        oc = pltpu.make_async_copy(bmps[cur], bmp_hbm.at[row0 + t],
                                   osems[cur])
        oc.start()
        pending_out[cur] = (oc, ci, msk)
    for cur in range(2):
        if pending_out[cur] is not None:
            pending_out[cur][0].wait()


def _out_body(bmp_ref, ncm_ref, gl_ref, gtb_ref,
              cls_ref, tb_ref, ts_ref, fm_ref, tg_ref, cfl_ref):
    b = pl.program_id(0)
    bs = pl.num_programs(0)
    ngt = gtb_ref.shape[1]
    na = bmp_ref.shape[1]

    @pl.when(b == 0)
    def _flags():
        acf = jnp.int32(0)
        for bb in range(bs):
            isb = bmp_ref[pl.ds(bb * ngt, ngt), :]
            cntb = jnp.sum(isb, axis=0, keepdims=True)
            acf = acf + jnp.max((cntb > 1).astype(jnp.int32))
        cfl_ref[0] = acf

    is_in = bmp_ref[pl.ds(b * ngt, ngt), :]
    cnt = jnp.sum(is_in, axis=0, keepdims=True)
    conflict = cnt > 1

    gl = gl_ref[0]
    gtb = gtb_ref[0]
    j_iota = jax.lax.broadcasted_iota(jnp.int32, (ngt, na), 0)
    ncm_row = ncm_ref[0][0:1, :]
    nm = jnp.logical_not(conflict).astype(jnp.int32)
    ncm = jnp.where(j_iota == 0, ncm_row, 0)
    resolved = (is_in + ncm) * nm
    anycf = cfl_ref[0]
    is_f = jnp.where(anycf > 0, resolved, is_in)

    fmask = jnp.sum(is_f, axis=0, keepdims=True) > 0
    mxv = jnp.max(is_f, axis=0, keepdims=True)
    tg = jnp.min(jnp.where(is_f == mxv, j_iota, ngt), axis=0,
                 keepdims=True)

    sel_t = j_iota == tg
    cmp_t = sel_t.astype(jnp.float32)
    glf = gl.astype(jnp.float32)
    tcls = jnp.sum(jnp.where(sel_t, glf, 0.0), axis=0, keepdims=True)

    cls_ref[0] = tcls.astype(jnp.int32)
    fm_ref[0] = fmask.astype(jnp.int32)
    tg_ref[0] = tg

    dn = (((0,), (0,)), ((), ()))
    tb_ref[0] = jax.lax.dot_general(cmp_t, gtb, dn,
                                    preferred_element_type=jnp.float32)
    oh_cls = (jax.lax.broadcasted_iota(jnp.int32, (ngt, NC), 1)
              == gl).astype(jnp.float32)
    ts_ref[0] = jax.lax.dot_general(cmp_t, oh_cls, dn,
                                    preferred_element_type=jnp.float32)


@jax.jit
def kernel(pd_scores, pd_bboxes, anc_points, gt_labels, gt_bboxes, gt_mask):
    bs, na, nc = pd_scores.shape
    ngt = gt_bboxes.shape[1]

    ps_s = pd_scores[:, :ngt, :]
    pb_t = jnp.transpose(pd_bboxes, (0, 2, 1))
    anc_t = jnp.transpose(anc_points).reshape(1, 2, na)
    gl = gt_labels.astype(jnp.int32)

    align3, ncm3 = pl.pallas_call(
        _align_body,
        grid=(bs,),
        in_specs=[
            pl.BlockSpec((1, ngt, nc), lambda b: (b, 0, 0)),
            pl.BlockSpec((1, 4, na), lambda b: (b, 0, 0)),
            pl.BlockSpec((1, 2, na), lambda b: (0, 0, 0)),
            pl.BlockSpec((1, ngt, 1), lambda b: (b, 0, 0)),
            pl.BlockSpec((1, ngt, 4), lambda b: (b, 0, 0)),
            pl.BlockSpec((1, ngt, 1), lambda b: (b, 0, 0)),
        ],
        out_specs=(
            pl.BlockSpec((1, ngt, na), lambda b: (b, 0, 0)),
            pl.BlockSpec((1, 8, na), lambda b: (b, 0, 0)),
        ),
        out_shape=(
            jax.ShapeDtypeStruct((bs, ngt, na), jnp.float32),
            jax.ShapeDtypeStruct((bs, 8, na), jnp.int32),
        ),
    )(ps_s, pb_t, anc_t, gl, gt_bboxes, gt_mask)

    mesh = plsc.VectorSubcoreMesh(core_axis_name="c", subcore_axis_name="s",
                                  num_cores=2, num_subcores=16)
    sc_topk = pl.kernel(
        _sc_topk_body,
        out_type=jax.ShapeDtypeStruct((bs * ngt, na), jnp.int32),
        mesh=mesh,
        compiler_params=pltpu.CompilerParams(needs_layout_passes=False),
        scratch_types=[
            pltpu.VMEM((na,), jnp.float32),
            pltpu.VMEM((na,), jnp.float32),
            pltpu.VMEM((na,), jnp.int32),
            pltpu.VMEM((na,), jnp.int32),
            pltpu.VMEM((na + 16,), jnp.float32),
            pltpu.VMEM((na + 16,), jnp.int32),
            pltpu.SemaphoreType.DMA,
            pltpu.SemaphoreType.DMA,
            pltpu.SemaphoreType.DMA,
            pltpu.SemaphoreType.DMA,
        ],
    )
    bmp = sc_topk(align3.reshape(bs * ngt, na))

    out_shapes = (
        jax.ShapeDtypeStruct((bs, 1, na), jnp.int32),
        jax.ShapeDtypeStruct((bs, na, 4), jnp.float32),
        jax.ShapeDtypeStruct((bs, na, NC), jnp.float32),
        jax.ShapeDtypeStruct((bs, 1, na), jnp.int32),
        jax.ShapeDtypeStruct((bs, 1, na), jnp.int32),
    )

    tcls, tb, ts, fm, tg = pl.pallas_call(
        _out_body,
        grid=(bs,),
        in_specs=[
            pl.BlockSpec((bs * ngt, na), lambda b: (0, 0)),
            pl.BlockSpec((1, 8, na), lambda b: (b, 0, 0)),
            pl.BlockSpec((1, ngt, 1), lambda b: (b, 0, 0)),
            pl.BlockSpec((1, ngt, 4), lambda b: (b, 0, 0)),
        ],
        out_specs=(
            pl.BlockSpec((1, 1, na), lambda b: (b, 0, 0)),
            pl.BlockSpec((1, na, 4), lambda b: (b, 0, 0)),
            pl.BlockSpec((1, na, NC), lambda b: (b, 0, 0)),
            pl.BlockSpec((1, 1, na), lambda b: (b, 0, 0)),
            pl.BlockSpec((1, 1, na), lambda b: (b, 0, 0)),
        ),
        out_shape=out_shapes,
        scratch_shapes=[
            pltpu.SMEM((1,), jnp.int32),
        ],
    )(bmp, ncm3, gl, gt_bboxes)

    return (tcls.reshape(bs, na), tb, ts,
            fm.reshape(bs, na) > 0, tg.reshape(bs, na))
